# baseline (speedup 1.0000x reference)
import functools

import jax
import jax.numpy as jnp
from jax import lax
from jax.experimental import pallas as pl
from jax.experimental.pallas import tpu as pltpu
from jax.experimental.pallas import tpu_sc as plsc

N = 4096
D = 128
E = 65536

NC = 2
NS = 16
L = 16


def _k1_body(x_ref, ws_w_ref, ws_b_ref, wo_w_ref, wo_b_ref, w_w_ref,
             conv_w_ref, conv_b_ref, ln1_g_ref, ln1_b_ref,
             sp_ref, o_ref, v_ref):
    x = x_ref[...]
    s = jnp.dot(x, ws_w_ref[...], preferred_element_type=jnp.float32) + ws_b_ref[...]
    sp_ref[...] = s * w_w_ref[...]
    o_ref[...] = jnp.dot(x, wo_w_ref[...], preferred_element_type=jnp.float32) + wo_b_ref[...]
    mu = jnp.mean(x, axis=-1, keepdims=True)
    var = jnp.mean((x - mu) ** 2, axis=-1, keepdims=True)
    ln = (x - mu) * lax.rsqrt(var + 1e-5) * ln1_g_ref[...] + ln1_b_ref[...]
    v = jnp.dot(ln, conv_w_ref[...], preferred_element_type=jnp.float32) + conv_b_ref[...]
    v_ref[...] = jnp.maximum(v, 0.0)


def _k1(x, ws_w, ws_b, wo_w, wo_b, w_w, conv_w, conv_b, ln1_g, ln1_b):
    blk = 512
    grid = (N // blk,)
    row_spec = pl.BlockSpec((blk, D), lambda i: (i, 0))
    full = pl.BlockSpec((D, D), lambda i: (0, 0))
    vec = pl.BlockSpec((1, D), lambda i: (0, 0))
    return pl.pallas_call(
        _k1_body,
        grid=grid,
        in_specs=[row_spec, full, vec, full, vec, vec, full, vec, vec, vec],
        out_specs=[row_spec, row_spec, row_spec],
        out_shape=[jax.ShapeDtypeStruct((N, D), jnp.float32)] * 3,
    )(x, ws_w, ws_b.reshape(1, D), wo_w, wo_b.reshape(1, D),
      w_w.reshape(1, D), conv_w, conv_b.reshape(1, D),
      ln1_g.reshape(1, D), ln1_b.reshape(1, D))


def _k4_body(a_ref, v_ref, x_ref, t1_w_ref, t1_b_ref, t2_w_ref, t2_b_ref,
             ln2_g_ref, ln2_b_ref,
             atten_ref, ctx_ref, ref_ref, *, blk):
    i = pl.program_id(0)
    a = a_ref[...]
    rows = i * blk + lax.broadcasted_iota(jnp.int32, (blk, N), 0)
    cols = lax.broadcasted_iota(jnp.int32, (blk, N), 1)
    a = jnp.where(rows == cols, a - 10000.0, a)
    m = jnp.max(a, axis=1, keepdims=True)
    p = jnp.exp(a - m)
    den = jnp.sum(p, axis=1, keepdims=True)
    atten = p / den
    atten_ref[...] = atten
    ctx = jnp.dot(atten, v_ref[...], preferred_element_type=jnp.float32)
    ctx_ref[...] = ctx
    out = x_ref[...] + ctx
    mu = jnp.mean(out, axis=-1, keepdims=True)
    var = jnp.mean((out - mu) ** 2, axis=-1, keepdims=True)
    ln = (out - mu) * lax.rsqrt(var + 1e-5) * ln2_g_ref[...] + ln2_b_ref[...]
    h = jnp.maximum(jnp.dot(ln, t1_w_ref[...], preferred_element_type=jnp.float32)
                    + t1_b_ref[...], 0.0)
    tr = jnp.dot(h, t2_w_ref[...], preferred_element_type=jnp.float32) + t2_b_ref[...]
    ref_ref[...] = jnp.maximum(out + tr, 0.0)


def _k4(a, v, x, t1_w, t1_b, t2_w, t2_b, ln2_g, ln2_b):
    blk = 256
    grid = (N // blk,)
    a_spec = pl.BlockSpec((blk, N), lambda i: (i, 0))
    row_spec = pl.BlockSpec((blk, D), lambda i: (i, 0))
    return pl.pallas_call(
        functools.partial(_k4_body, blk=blk),
        grid=grid,
        in_specs=[a_spec,
                  pl.BlockSpec((N, D), lambda i: (0, 0)),
                  row_spec,
                  pl.BlockSpec((D, 2 * D), lambda i: (0, 0)),
                  pl.BlockSpec((1, 2 * D), lambda i: (0, 0)),
                  pl.BlockSpec((2 * D, D), lambda i: (0, 0)),
                  pl.BlockSpec((1, D), lambda i: (0, 0)),
                  pl.BlockSpec((1, D), lambda i: (0, 0)),
                  pl.BlockSpec((1, D), lambda i: (0, 0))],
        out_specs=[a_spec, row_spec, row_spec],
        out_shape=[jax.ShapeDtypeStruct((N, N), jnp.float32),
                   jax.ShapeDtypeStruct((N, D), jnp.float32),
                   jax.ShapeDtypeStruct((N, D), jnp.float32)],
    )(a, v, x, t1_w, t1_b.reshape(1, 2 * D), t2_w, t2_b.reshape(1, D),
      ln2_g.reshape(1, D), ln2_b.reshape(1, D))


_CHUNK = 128
_EPW = E // (NC * NS)


def _k2_body(sp_hbm, o_hbm, u_hbm, src_hbm, dst_hbm, wb_hbm,
             af_hbm, flat_hbm,
             src_v, dst_v, s_v, o_v, u_v, af_v, flat_v, wb_v, t_st,
             sem1, sem2):
    wid = lax.axis_index("s") * NC + lax.axis_index("c")
    pltpu.sync_copy(wb_hbm, wb_v)
    wb = wb_v[...]

    def chunk_body(ci, _):
        base = wid * _EPW + ci * _CHUNK
        pltpu.sync_copy(src_hbm.at[pl.ds(base, _CHUNK)], src_v)
        pltpu.sync_copy(dst_hbm.at[pl.ds(base, _CHUNK)], dst_v)
        cp1 = pltpu.async_copy(sp_hbm.at[src_v], s_v, sem1)
        cp2 = pltpu.async_copy(o_hbm.at[dst_v], o_v, sem2)
        pltpu.sync_copy(u_hbm.at[pl.ds(base, _CHUNK)], u_v)
        cp1.wait()
        cp2.wait()

        def flat_body(j, _):
            sv = src_v[pl.ds(j * L, L)]
            dv = dst_v[pl.ds(j * L, L)]
            flat_v[pl.ds(j * L, L)] = sv * N + dv
            return 0

        lax.fori_loop(0, _CHUNK // L, flat_body, 0, unroll=4)

        lane = lax.iota(jnp.int32, L)

        def group_body(g, _):
            def edge_body(ee, res):
                e = g * L + ee
                acc = (s_v[e, pl.ds(0, L)] * o_v[e, pl.ds(0, L)]
                       * u_v[e, pl.ds(0, L)])
                for k in range(1, D // L):
                    acc = acc + (s_v[e, pl.ds(k * L, L)]
                                 * o_v[e, pl.ds(k * L, L)]
                                 * u_v[e, pl.ds(k * L, L)])
                return jnp.where(lane == ee, jnp.sum(acc), res)

            res = lax.fori_loop(0, L, edge_body, jnp.zeros((L,), jnp.float32),
                                unroll=2)
            af_v[pl.ds(g * L, L)] = res + wb
            return 0

        lax.fori_loop(0, _CHUNK // L, group_body, 0)
        pltpu.sync_copy(af_v, af_hbm.at[pl.ds(base, _CHUNK)])
        pltpu.sync_copy(flat_v, flat_hbm.at[pl.ds(base, _CHUNK)])
        return 0

    lax.fori_loop(0, _EPW // _CHUNK, chunk_body, 0)


def _k2(sp, o, u, src, dst, wb_vec):
    mesh = plsc.VectorSubcoreMesh(core_axis_name="c", subcore_axis_name="s",
                                  num_cores=NC, num_subcores=NS)
    return pl.kernel(
        _k2_body,
        out_type=[jax.ShapeDtypeStruct((E,), jnp.float32),
                  jax.ShapeDtypeStruct((E,), jnp.int32)],
        mesh=mesh,
        compiler_params=pltpu.CompilerParams(needs_layout_passes=False),
        scratch_types=[
            pltpu.VMEM((_CHUNK,), jnp.int32),
            pltpu.VMEM((_CHUNK,), jnp.int32),
            pltpu.VMEM((_CHUNK, D), jnp.float32),
            pltpu.VMEM((_CHUNK, D), jnp.float32),
            pltpu.VMEM((_CHUNK, D), jnp.float32),
            pltpu.VMEM((_CHUNK,), jnp.float32),
            pltpu.VMEM((_CHUNK,), jnp.int32),
            pltpu.VMEM((L,), jnp.float32),
            pltpu.VMEM((L * L,), jnp.float32),
            pltpu.SemaphoreType.DMA,
            pltpu.SemaphoreType.DMA,
        ],
    )(sp, o, u, src, dst, wb_vec)


_SLAB_ROWS = 256
_NSLABS = N // _SLAB_ROWS
_SLAB_W = _SLAB_ROWS * N
_EPS = E // NS
_ZCH = 2048


def _k3_body(flat_hbm, af_hbm, a_hbm,
             flat_v, af_v, idx_st, val_st, zeros_v, slab, sem):
    cid = lax.axis_index("c")
    sid = lax.axis_index("s")

    def zb(j, _):
        zeros_v[pl.ds(j * L, L)] = jnp.zeros((L,), jnp.float32)
        return 0

    lax.fori_loop(0, _ZCH // L, zb, 0, unroll=8)
    pltpu.sync_copy(flat_hbm.at[pl.ds(sid * _EPS, _EPS)], flat_v)
    pltpu.sync_copy(af_hbm.at[pl.ds(sid * _EPS, _EPS)], af_v)

    def slab_body(t, _):
        slab_id = t * NC + cid
        slab_base = slab_id * _SLAB_W

        n_z = _SLAB_W // NS // _ZCH
        def zc(j, _):
            pltpu.sync_copy(zeros_v,
                            slab.at[pl.ds((sid * n_z + j) * _ZCH, _ZCH)])
            return 0
        lax.fori_loop(0, n_z, zc, 0)
        @pl.when(sid == 0)
        def _():
            pltpu.sync_copy(zeros_v.at[pl.ds(0, 8)], slab.at[pl.ds(_SLAB_W, 8)])
        plsc.subcore_barrier()

        def stage(i, _):
            fl = flat_v[pl.ds(i * L, L)]
            av = af_v[pl.ds(i * L, L)]
            loc = fl - slab_base
            m = (loc >= 0) & (loc < _SLAB_W)
            j = i // 8
            c = (i % 8) * L
            idx_st[j, pl.ds(c, L)] = jnp.where(m, loc, _SLAB_W)
            val_st[j, pl.ds(c, L)] = jnp.where(m, av, 0.0)
            return 0

        lax.fori_loop(0, _EPS // L, stage, 0, unroll=4)

        def dma_row(j, _):
            pltpu.sync_copy(val_st.at[j], slab.at[idx_st.at[j]], add=True)
            return 0

        lax.fori_loop(0, _EPS // 128, dma_row, 0)
        plsc.subcore_barrier()

        w = _SLAB_W // NS
        pltpu.sync_copy(slab.at[pl.ds(sid * w, w)],
                        a_hbm.at[pl.ds(slab_base + sid * w, w)])
        plsc.subcore_barrier()
        return 0

    lax.fori_loop(0, _NSLABS // NC, slab_body, 0)


def _k3(flat, af):
    mesh = plsc.VectorSubcoreMesh(core_axis_name="c", subcore_axis_name="s",
                                  num_cores=NC, num_subcores=NS)
    return pl.kernel(
        _k3_body,
        out_type=jax.ShapeDtypeStruct((N * N,), jnp.float32),
        mesh=mesh,
        compiler_params=pltpu.CompilerParams(needs_layout_passes=False),
        scratch_types=[
            pltpu.VMEM((_EPS,), jnp.int32),
            pltpu.VMEM((_EPS,), jnp.float32),
            pltpu.VMEM((_EPS // 128, 128), jnp.int32),
            pltpu.VMEM((_EPS // 128, 128), jnp.float32),
            pltpu.VMEM((_ZCH,), jnp.float32),
            pltpu.VMEM_SHARED((_SLAB_W + 8,), jnp.float32),
            pltpu.SemaphoreType.DMA,
        ],
    )(flat, af)


def kernel(obj_feats, phr_feats, pair_idxs, ws_w, ws_b, wo_w, wo_b, w_w, w_b,
           conv_w, conv_b, t1_w, t1_b, t2_w, t2_b, ln1_g, ln1_b, ln2_g, ln2_b):
    x = obj_feats[0]
    u = phr_feats[0]
    pi = pair_idxs[0].astype(jnp.int32)
    src = pi[:, 0]
    dst = pi[:, 1]

    sp, o, v = _k1(x, ws_w, ws_b, wo_w, wo_b, w_w, conv_w, conv_b, ln1_g, ln1_b)
    wb_vec = jnp.full((L,), w_b[0], jnp.float32)
    af, flat = _k2(sp, o, u, src, dst, wb_vec)
    a = _k3(flat, af).reshape(N, N)
    atten, ctx, refined = _k4(a, v, x, t1_w, t1_b, t2_w, t2_b, ln2_g, ln2_b)
    return refined, atten, ctx


# speedup vs baseline: 2.3845x; 2.3845x over previous
import functools

import jax
import jax.numpy as jnp
from jax import lax
from jax.experimental import pallas as pl
from jax.experimental.pallas import tpu as pltpu
from jax.experimental.pallas import tpu_sc as plsc

N = 4096
D = 128
E = 65536

NC = 2
NS = 16
L = 16


def _k1_body(x_ref, ws_w_ref, ws_b_ref, wo_w_ref, wo_b_ref, w_w_ref,
             conv_w_ref, conv_b_ref, ln1_g_ref, ln1_b_ref,
             sp_ref, o_ref, v_ref):
    x = x_ref[...]
    s = jnp.dot(x, ws_w_ref[...], preferred_element_type=jnp.float32) + ws_b_ref[...]
    sp_ref[...] = s * w_w_ref[...]
    o_ref[...] = jnp.dot(x, wo_w_ref[...], preferred_element_type=jnp.float32) + wo_b_ref[...]
    mu = jnp.mean(x, axis=-1, keepdims=True)
    var = jnp.mean((x - mu) ** 2, axis=-1, keepdims=True)
    ln = (x - mu) * lax.rsqrt(var + 1e-5) * ln1_g_ref[...] + ln1_b_ref[...]
    v = jnp.dot(ln, conv_w_ref[...], preferred_element_type=jnp.float32) + conv_b_ref[...]
    v_ref[...] = jnp.maximum(v, 0.0)


def _k1(x, ws_w, ws_b, wo_w, wo_b, w_w, conv_w, conv_b, ln1_g, ln1_b):
    blk = 512
    grid = (N // blk,)
    row_spec = pl.BlockSpec((blk, D), lambda i: (i, 0))
    full = pl.BlockSpec((D, D), lambda i: (0, 0))
    vec = pl.BlockSpec((1, D), lambda i: (0, 0))
    return pl.pallas_call(
        _k1_body,
        grid=grid,
        in_specs=[row_spec, full, vec, full, vec, vec, full, vec, vec, vec],
        out_specs=[row_spec, row_spec, row_spec],
        out_shape=[jax.ShapeDtypeStruct((N, D), jnp.float32)] * 3,
    )(x, ws_w, ws_b.reshape(1, D), wo_w, wo_b.reshape(1, D),
      w_w.reshape(1, D), conv_w, conv_b.reshape(1, D),
      ln1_g.reshape(1, D), ln1_b.reshape(1, D))


def _k4_body(a_ref, v_ref, x_ref, t1_w_ref, t1_b_ref, t2_w_ref, t2_b_ref,
             ln2_g_ref, ln2_b_ref,
             atten_ref, ctx_ref, ref_ref, *, blk):
    i = pl.program_id(0)
    a = a_ref[...]
    rows = i * blk + lax.broadcasted_iota(jnp.int32, (blk, N), 0)
    cols = lax.broadcasted_iota(jnp.int32, (blk, N), 1)
    a = jnp.where(rows == cols, a - 10000.0, a)
    m = jnp.max(a, axis=1, keepdims=True)
    p = jnp.exp(a - m)
    den = jnp.sum(p, axis=1, keepdims=True)
    atten = p / den
    atten_ref[...] = atten
    ctx = jnp.dot(atten, v_ref[...], preferred_element_type=jnp.float32)
    ctx_ref[...] = ctx
    out = x_ref[...] + ctx
    mu = jnp.mean(out, axis=-1, keepdims=True)
    var = jnp.mean((out - mu) ** 2, axis=-1, keepdims=True)
    ln = (out - mu) * lax.rsqrt(var + 1e-5) * ln2_g_ref[...] + ln2_b_ref[...]
    h = jnp.maximum(jnp.dot(ln, t1_w_ref[...], preferred_element_type=jnp.float32)
                    + t1_b_ref[...], 0.0)
    tr = jnp.dot(h, t2_w_ref[...], preferred_element_type=jnp.float32) + t2_b_ref[...]
    ref_ref[...] = jnp.maximum(out + tr, 0.0)


def _k4(a, v, x, t1_w, t1_b, t2_w, t2_b, ln2_g, ln2_b):
    blk = 256
    grid = (N // blk,)
    a_spec = pl.BlockSpec((blk, N), lambda i: (i, 0))
    row_spec = pl.BlockSpec((blk, D), lambda i: (i, 0))
    return pl.pallas_call(
        functools.partial(_k4_body, blk=blk),
        grid=grid,
        in_specs=[a_spec,
                  pl.BlockSpec((N, D), lambda i: (0, 0)),
                  row_spec,
                  pl.BlockSpec((D, 2 * D), lambda i: (0, 0)),
                  pl.BlockSpec((1, 2 * D), lambda i: (0, 0)),
                  pl.BlockSpec((2 * D, D), lambda i: (0, 0)),
                  pl.BlockSpec((1, D), lambda i: (0, 0)),
                  pl.BlockSpec((1, D), lambda i: (0, 0)),
                  pl.BlockSpec((1, D), lambda i: (0, 0))],
        out_specs=[a_spec, row_spec, row_spec],
        out_shape=[jax.ShapeDtypeStruct((N, N), jnp.float32),
                   jax.ShapeDtypeStruct((N, D), jnp.float32),
                   jax.ShapeDtypeStruct((N, D), jnp.float32)],
    )(a, v, x, t1_w, t1_b.reshape(1, 2 * D), t2_w, t2_b.reshape(1, D),
      ln2_g.reshape(1, D), ln2_b.reshape(1, D))


_CHUNK = 128
_EPW = E // (NC * NS)


def _k2_body(sp_hbm, o_hbm, u_hbm, src_hbm, dst_hbm, wb_hbm,
             af_hbm, flat_hbm,
             src_v, dst_v, s_v, o_v, u_v, af_v, flat_v, wb_v, t_st,
             sem1, sem2):
    wid = lax.axis_index("s") * NC + lax.axis_index("c")
    pltpu.sync_copy(wb_hbm, wb_v)
    wb = wb_v[...]

    def chunk_body(ci, _):
        base = wid * _EPW + ci * _CHUNK
        pltpu.sync_copy(src_hbm.at[pl.ds(base, _CHUNK)], src_v)
        pltpu.sync_copy(dst_hbm.at[pl.ds(base, _CHUNK)], dst_v)
        cp1 = pltpu.async_copy(sp_hbm.at[src_v], s_v, sem1)
        cp2 = pltpu.async_copy(o_hbm.at[dst_v], o_v, sem2)
        pltpu.sync_copy(u_hbm.at[pl.ds(base, _CHUNK)], u_v)
        cp1.wait()
        cp2.wait()

        def flat_body(j, _):
            sv = src_v[pl.ds(j * L, L)]
            dv = dst_v[pl.ds(j * L, L)]
            flat_v[pl.ds(j * L, L)] = sv * N + dv
            return 0

        lax.fori_loop(0, _CHUNK // L, flat_body, 0, unroll=4)

        lane = lax.iota(jnp.int32, L)

        def group_body(g, _):
            def edge_body(ee, res):
                e = g * L + ee
                acc = (s_v[e, pl.ds(0, L)] * o_v[e, pl.ds(0, L)]
                       * u_v[e, pl.ds(0, L)])
                for k in range(1, D // L):
                    acc = acc + (s_v[e, pl.ds(k * L, L)]
                                 * o_v[e, pl.ds(k * L, L)]
                                 * u_v[e, pl.ds(k * L, L)])
                return jnp.where(lane == ee, jnp.sum(acc), res)

            res = lax.fori_loop(0, L, edge_body, jnp.zeros((L,), jnp.float32),
                                unroll=2)
            af_v[pl.ds(g * L, L)] = res + wb
            return 0

        lax.fori_loop(0, _CHUNK // L, group_body, 0)
        pltpu.sync_copy(af_v, af_hbm.at[pl.ds(base, _CHUNK)])
        pltpu.sync_copy(flat_v, flat_hbm.at[pl.ds(base, _CHUNK)])
        return 0

    lax.fori_loop(0, _EPW // _CHUNK, chunk_body, 0)


def _k2(sp, o, u, src, dst, wb_vec):
    mesh = plsc.VectorSubcoreMesh(core_axis_name="c", subcore_axis_name="s",
                                  num_cores=NC, num_subcores=NS)
    return pl.kernel(
        _k2_body,
        out_type=[jax.ShapeDtypeStruct((E,), jnp.float32),
                  jax.ShapeDtypeStruct((E,), jnp.int32)],
        mesh=mesh,
        compiler_params=pltpu.CompilerParams(needs_layout_passes=False),
        scratch_types=[
            pltpu.VMEM((_CHUNK,), jnp.int32),
            pltpu.VMEM((_CHUNK,), jnp.int32),
            pltpu.VMEM((_CHUNK, D), jnp.float32),
            pltpu.VMEM((_CHUNK, D), jnp.float32),
            pltpu.VMEM((_CHUNK, D), jnp.float32),
            pltpu.VMEM((_CHUNK,), jnp.float32),
            pltpu.VMEM((_CHUNK,), jnp.int32),
            pltpu.VMEM((L,), jnp.float32),
            pltpu.VMEM((L * L,), jnp.float32),
            pltpu.SemaphoreType.DMA,
            pltpu.SemaphoreType.DMA,
        ],
    )(sp, o, u, src, dst, wb_vec)


_SLAB_ROWS = 256
_NSLABS = N // _SLAB_ROWS
_SLAB_W = _SLAB_ROWS * N
_EPS = E // NS
_ZCH = 2048


def _k3_body(flat_hbm, af_hbm, a_hbm,
             flat_v, af_v, idx_st, val_st, zeros_v, slab, sem):
    cid = lax.axis_index("c")
    sid = lax.axis_index("s")

    def zb(j, _):
        zeros_v[pl.ds(j * L, L)] = jnp.zeros((L,), jnp.float32)
        return 0

    lax.fori_loop(0, _ZCH // L, zb, 0, unroll=8)
    pltpu.sync_copy(flat_hbm.at[pl.ds(sid * _EPS, _EPS)], flat_v)
    pltpu.sync_copy(af_hbm.at[pl.ds(sid * _EPS, _EPS)], af_v)

    def slab_body(t, _):
        slab_id = t * NC + cid
        slab_base = slab_id * _SLAB_W

        n_z = _SLAB_W // NS // _ZCH
        def zc(j, _):
            pltpu.sync_copy(zeros_v,
                            slab.at[pl.ds((sid * n_z + j) * _ZCH, _ZCH)])
            return 0
        lax.fori_loop(0, n_z, zc, 0)
        @pl.when(sid == 0)
        def _():
            pltpu.sync_copy(zeros_v.at[pl.ds(0, 8)], slab.at[pl.ds(_SLAB_W, 8)])
        plsc.subcore_barrier()

        def stage(i, cnt):
            fl = flat_v[pl.ds(i * L, L)]
            av = af_v[pl.ds(i * L, L)]
            loc = fl - slab_base
            m = (loc >= 0) & (loc < _SLAB_W)
            plsc.store_compressed(idx_st.at[pl.ds(cnt, L)], loc, mask=m)
            plsc.store_compressed(val_st.at[pl.ds(cnt, L)], av, mask=m)
            return cnt + plsc.all_reduce_population_count(m)[0]

        cnt = lax.fori_loop(0, _EPS // L, stage, jnp.int32(0), unroll=4)
        idx_st[pl.ds(cnt, L)] = jnp.full((L,), _SLAB_W, jnp.int32)
        val_st[pl.ds(cnt, L)] = jnp.zeros((L,), jnp.float32)

        def dma_chunk(j, _):
            @pl.when(j * L < cnt)
            def _():
                iv = idx_st[pl.ds(j * L, L)]
                pltpu.sync_copy(val_st.at[pl.ds(j * L, L)],
                                slab.at[iv], add=True)
            return 0

        lax.fori_loop(0, _EPS // L, dma_chunk, 0)
        plsc.subcore_barrier()

        w = _SLAB_W // NS
        pltpu.sync_copy(slab.at[pl.ds(sid * w, w)],
                        a_hbm.at[pl.ds(slab_base + sid * w, w)])
        plsc.subcore_barrier()
        return 0

    lax.fori_loop(0, _NSLABS // NC, slab_body, 0)


def _k3(flat, af):
    mesh = plsc.VectorSubcoreMesh(core_axis_name="c", subcore_axis_name="s",
                                  num_cores=NC, num_subcores=NS)
    return pl.kernel(
        _k3_body,
        out_type=jax.ShapeDtypeStruct((N * N,), jnp.float32),
        mesh=mesh,
        compiler_params=pltpu.CompilerParams(needs_layout_passes=False),
        scratch_types=[
            pltpu.VMEM((_EPS,), jnp.int32),
            pltpu.VMEM((_EPS,), jnp.float32),
            pltpu.VMEM((_EPS + 2 * L,), jnp.int32),
            pltpu.VMEM((_EPS + 2 * L,), jnp.float32),
            pltpu.VMEM((_ZCH,), jnp.float32),
            pltpu.VMEM_SHARED((_SLAB_W + 8,), jnp.float32),
            pltpu.SemaphoreType.DMA,
        ],
    )(flat, af)


def kernel(obj_feats, phr_feats, pair_idxs, ws_w, ws_b, wo_w, wo_b, w_w, w_b,
           conv_w, conv_b, t1_w, t1_b, t2_w, t2_b, ln1_g, ln1_b, ln2_g, ln2_b):
    x = obj_feats[0]
    u = phr_feats[0]
    pi = pair_idxs[0].astype(jnp.int32)
    src = pi[:, 0]
    dst = pi[:, 1]

    sp, o, v = _k1(x, ws_w, ws_b, wo_w, wo_b, w_w, conv_w, conv_b, ln1_g, ln1_b)
    wb_vec = jnp.full((L,), w_b[0], jnp.float32)
    af, flat = _k2(sp, o, u, src, dst, wb_vec)
    a = _k3(flat, af).reshape(N, N)
    atten, ctx, refined = _k4(a, v, x, t1_w, t1_b, t2_w, t2_b, ln2_g, ln2_b)
    return refined, atten, ctx
